# baseline (speedup 1.0000x reference)
import functools

import jax
import jax.numpy as jnp
from jax import lax
from jax.experimental import pallas as pl
from jax.experimental.pallas import tpu as pltpu
from jax.experimental.pallas import tpu_sc as plsc

NC = 2
NS = 16
NW = NC * NS
D = 128
BLK = 512
CHUNK = 128


def _sc_mesh():
    return plsc.VectorSubcoreMesh(core_axis_name="c", subcore_axis_name="s")


def _make_deg_kernel(n_pad, jpt):
    rpt = n_pad // NS

    @functools.partial(
        pl.kernel,
        out_type=jax.ShapeDtypeStruct((NC, n_pad), jnp.float32),
        mesh=_sc_mesh(),
        scratch_types=[
            pltpu.VMEM((jpt, CHUNK), jnp.int32),
            pltpu.VMEM((CHUNK,), jnp.float32),
            pltpu.VMEM((rpt,), jnp.float32),
            pltpu.VMEM_SHARED((n_pad,), jnp.float32),
        ],
    )
    def deg_kernel(dst2d, out, idx_v, ones_v, zeros_v, shared):
        c = lax.axis_index("c")
        s = lax.axis_index("s")
        wid = c * NS + s
        pltpu.sync_copy(dst2d.at[pl.ds(wid * jpt, jpt)], idx_v)

        def fill_ones(i, _):
            ones_v[pl.ds(i * 16, 16)] = jnp.full((16,), 1.0, jnp.float32)
            return 0

        lax.fori_loop(0, CHUNK // 16, fill_ones, 0)

        def fill_zeros(i, _):
            zeros_v[pl.ds(i * 16, 16)] = jnp.zeros((16,), jnp.float32)
            return 0

        lax.fori_loop(0, rpt // 16, fill_zeros, 0)
        pltpu.sync_copy(zeros_v, shared.at[pl.ds(s * rpt, rpt)])
        plsc.subcore_barrier()

        def body(j, _):
            pltpu.sync_copy(ones_v, shared.at[idx_v.at[j]], add=True)
            return 0

        lax.fori_loop(0, jpt, body, 0)
        plsc.subcore_barrier()
        pltpu.sync_copy(shared.at[pl.ds(s * rpt, rpt)],
                        out.at[c, pl.ds(s * rpt, rpt)])

    return deg_kernel


def _make_agg_kernel(n_pad, jpt, nw):
    rpt = n_pad // NS

    H = jpt * 2 // nw
    assert H % 8 == 0 and H >= 4

    @functools.partial(
        pl.kernel,
        out_type=jax.ShapeDtypeStruct((1, n_pad, D), jnp.float32),
        mesh=plsc.VectorSubcoreMesh(core_axis_name="c",
                                    subcore_axis_name="s", num_cores=1),
        scratch_types=[
            pltpu.VMEM((H, CHUNK), jnp.int32),
            pltpu.VMEM((H, CHUNK), jnp.int32),
            pltpu.VMEM((CHUNK, D), jnp.float32),
            pltpu.VMEM((CHUNK, D), jnp.float32),
            pltpu.VMEM_SHARED((n_pad, D), jnp.float32),
            pltpu.SemaphoreType.DMA,
            pltpu.SemaphoreType.DMA,
            pltpu.SemaphoreType.DMA,
        ],
    )
    def agg_kernel(m_hbm, src2d, dst2d, out, sidx, didx, buf0, buf1,
                   shared, sem, gsem, ssem):
        bufs = (buf0, buf1)
        s = lax.axis_index("s")
        wbase = s * nw

        def load_idx(w):
            base = (wbase + w) * H
            pltpu.async_copy(src2d.at[pl.ds(base, H)], sidx, sem)
            pltpu.async_copy(dst2d.at[pl.ds(base, H)], didx, sem)
            pltpu.make_async_copy(src2d.at[pl.ds(base, H)], sidx, sem).wait()
            pltpu.make_async_copy(dst2d.at[pl.ds(base, H)], didx, sem).wait()

        def start_g(j, r):
            pltpu.async_copy(m_hbm.at[sidx.at[j]], bufs[r], gsem)

        def start_s(j, r):
            pltpu.async_copy(bufs[r], shared.at[didx.at[j]], ssem, add=True)

        def wait_g(j, r):
            pltpu.make_async_copy(m_hbm.at[sidx.at[j]], bufs[r], gsem).wait()

        def wait_s(j, r):
            pltpu.make_async_copy(bufs[r], shared.at[didx.at[j]],
                                  ssem).wait()

        cp_i = pltpu.async_copy(m_hbm.at[pl.ds(s * rpt, rpt)],
                                shared.at[pl.ds(s * rpt, rpt)], gsem)
        load_idx(0)
        cp_i.wait()
        plsc.subcore_barrier()

        def body(g, _):
            for r in (0, 1):
                j = 2 * g + r
                wait_g(j, r)
                start_s(j, r)
                wait_s(j - 1, 1 - r)

                @pl.when(j + 1 < H)
                def _():
                    start_g(j + 1, 1 - r)

            return 0

        def run_window(w, _):
            @pl.when(w > 0)
            def _():
                load_idx(w)

            start_g(0, 0)
            wait_g(0, 0)
            start_s(0, 0)
            start_g(1, 1)
            wait_g(1, 1)
            start_s(1, 1)
            wait_s(0, 0)
            start_g(2, 0)
            lax.fori_loop(1, H // 2, body, 0)
            wait_s(H - 1, 1)
            return 0

        lax.fori_loop(0, nw, run_window, 0)
        plsc.subcore_barrier()
        pltpu.sync_copy(shared.at[pl.ds(s * rpt, rpt)],
                        out.at[0, pl.ds(s * rpt, rpt)])

    return agg_kernel


def _bdot(a, b):
    return jnp.dot(a.astype(jnp.bfloat16), b.astype(jnp.bfloat16),
                   preferred_element_type=jnp.float32)


def _dinv_from(degp_ref):
    deg = degp_ref[0, :] + degp_ref[1, :] + 1.0
    return lax.rsqrt(deg)


def _tc_first(degp, xp, W0, b0, W1):
    n_pad = xp.shape[0]

    def body(degp_ref, x_ref, w0_ref, b0_ref, w1_ref, m_ref):
        dinv = _dinv_from(degp_ref)
        h0 = _bdot(x_ref[...], w0_ref[...]) + b0_ref[...][None, :]
        m_ref[...] = dinv[:, None] * _bdot(h0, w1_ref[...])

    return pl.pallas_call(
        body,
        grid=(n_pad // BLK,),
        in_specs=[
            pl.BlockSpec((2, BLK), lambda i: (0, i)),
            pl.BlockSpec((BLK, D), lambda i: (i, 0)),
            pl.BlockSpec((D, D), lambda i: (0, 0)),
            pl.BlockSpec((D,), lambda i: (0,)),
            pl.BlockSpec((D, D), lambda i: (0, 0)),
        ],
        out_specs=pl.BlockSpec((BLK, D), lambda i: (i, 0)),
        out_shape=jax.ShapeDtypeStruct((n_pad, D), jnp.float32),
    )(degp, xp, W0, b0, W1)


def _tc_mid(degp, part, b_prev, W_next):
    n_pad = part.shape[1]

    def body(degp_ref, p_ref, b_ref, w_ref, o_ref):
        dinv = _dinv_from(degp_ref)
        h = jax.nn.relu(dinv[:, None] * p_ref[0] + b_ref[...][None, :])
        o_ref[...] = dinv[:, None] * _bdot(h, w_ref[...])

    return pl.pallas_call(
        body,
        grid=(n_pad // BLK,),
        in_specs=[
            pl.BlockSpec((2, BLK), lambda i: (0, i)),
            pl.BlockSpec((1, BLK, D), lambda i: (0, i, 0)),
            pl.BlockSpec((D,), lambda i: (0,)),
            pl.BlockSpec((D, D), lambda i: (0, 0)),
        ],
        out_specs=pl.BlockSpec((BLK, D), lambda i: (i, 0)),
        out_shape=jax.ShapeDtypeStruct((n_pad, D), jnp.float32),
    )(degp, part, b_prev, W_next)


def _tc_final(degp, part, b3, wf_row, bf2):
    n_pad = part.shape[1]

    def body(degp_ref, p_ref, b_ref, wf_ref, bf_ref, o_ref):
        dinv = _dinv_from(degp_ref)
        h3 = dinv[:, None] * p_ref[0] + b_ref[...][None, :]
        h3r = h3.astype(jnp.bfloat16).astype(jnp.float32)
        wfr = wf_ref[...].astype(jnp.bfloat16).astype(jnp.float32)
        o_ref[...] = jnp.sum(h3r * wfr, axis=1) + bf_ref[0]

    return pl.pallas_call(
        body,
        grid=(n_pad // BLK,),
        in_specs=[
            pl.BlockSpec((2, BLK), lambda i: (0, i)),
            pl.BlockSpec((1, BLK, D), lambda i: (0, i, 0)),
            pl.BlockSpec((D,), lambda i: (0,)),
            pl.BlockSpec((1, D), lambda i: (0, 0)),
            pl.BlockSpec(memory_space=pltpu.MemorySpace.SMEM),
        ],
        out_specs=pl.BlockSpec((BLK,), lambda i: (i,)),
        out_shape=jax.ShapeDtypeStruct((n_pad,), jnp.float32),
    )(degp, part, b3, wf_row, bf2)


def kernel(x, edge_index, W0, b0, W1, b1, W2, b2, W3, b3, Wf, bf):
    n = x.shape[0]
    e = edge_index.shape[1]
    n_pad = ((n + BLK - 1) // BLK) * BLK
    gran = NW * CHUNK * 8
    e_pad = ((e + gran - 1) // gran) * gran
    jpt = e_pad // (NW * CHUNK)

    npad_ids = jnp.arange(e_pad - e, dtype=jnp.int32)
    pad_src = jnp.full((e_pad - e,), n, jnp.int32)
    pad_dst = n + (npad_ids % (n_pad - n))
    src2d = jnp.concatenate([edge_index[0], pad_src]).reshape(
        e_pad // CHUNK, CHUNK)
    dst2d = jnp.concatenate([edge_index[1], pad_dst]).reshape(
        e_pad // CHUNK, CHUNK)
    xp = jnp.pad(x, ((0, n_pad - n), (0, 0)))
    wf_row = Wf.reshape(1, D)
    bf2 = bf.reshape(1)

    deg_k = _make_deg_kernel(n_pad, jpt)
    agg_k = _make_agg_kernel(n_pad, jpt, 4)

    degp = deg_k(dst2d)
    m1 = _tc_first(degp, xp, W0, b0, W1)
    p1 = agg_k(m1, src2d, dst2d)
    m2 = _tc_mid(degp, p1, b1, W2)
    p2 = agg_k(m2, src2d, dst2d)
    m3 = _tc_mid(degp, p2, b2, W3)
    p3 = agg_k(m3, src2d, dst2d)
    outp = _tc_final(degp, p3, b3, wf_row, bf2)
    return outp[:n]


# speedup vs baseline: 1.0080x; 1.0080x over previous
import functools

import jax
import jax.numpy as jnp
from jax import lax
from jax.experimental import pallas as pl
from jax.experimental.pallas import tpu as pltpu
from jax.experimental.pallas import tpu_sc as plsc

NC = 2
NS = 16
NW = NC * NS
D = 128
BLK = 512
CHUNK = 128


def _sc_mesh():
    return plsc.VectorSubcoreMesh(core_axis_name="c", subcore_axis_name="s")


def _make_deg_kernel(n_pad, jpt):
    rpt = n_pad // NS

    @functools.partial(
        pl.kernel,
        out_type=jax.ShapeDtypeStruct((NC, n_pad), jnp.float32),
        mesh=_sc_mesh(),
        scratch_types=[
            pltpu.VMEM((jpt, CHUNK), jnp.int32),
            pltpu.VMEM((CHUNK,), jnp.float32),
            pltpu.VMEM((rpt,), jnp.float32),
            pltpu.VMEM_SHARED((n_pad,), jnp.float32),
        ],
    )
    def deg_kernel(dst2d, out, idx_v, ones_v, zeros_v, shared):
        c = lax.axis_index("c")
        s = lax.axis_index("s")
        wid = c * NS + s
        pltpu.sync_copy(dst2d.at[pl.ds(wid * jpt, jpt)], idx_v)

        def fill_ones(i, _):
            ones_v[pl.ds(i * 16, 16)] = jnp.full((16,), 1.0, jnp.float32)
            return 0

        lax.fori_loop(0, CHUNK // 16, fill_ones, 0)

        def fill_zeros(i, _):
            zeros_v[pl.ds(i * 16, 16)] = jnp.zeros((16,), jnp.float32)
            return 0

        lax.fori_loop(0, rpt // 16, fill_zeros, 0)
        pltpu.sync_copy(zeros_v, shared.at[pl.ds(s * rpt, rpt)])
        plsc.subcore_barrier()

        def body(j, _):
            pltpu.sync_copy(ones_v, shared.at[idx_v.at[j]], add=True)
            return 0

        lax.fori_loop(0, jpt, body, 0)
        plsc.subcore_barrier()
        pltpu.sync_copy(shared.at[pl.ds(s * rpt, rpt)],
                        out.at[c, pl.ds(s * rpt, rpt)])

    return deg_kernel


def _make_agg_kernel(n_pad, jpt, nw):
    rpt = n_pad // NS

    H = jpt * 2 // nw
    assert H % 8 == 0 and H >= 4

    @functools.partial(
        pl.kernel,
        out_type=jax.ShapeDtypeStruct((1, n_pad, D), jnp.float32),
        mesh=_sc_mesh(),
        scratch_types=[
            pltpu.VMEM((H, CHUNK), jnp.int32),
            pltpu.VMEM((H, CHUNK), jnp.int32),
            pltpu.VMEM((CHUNK, D), jnp.float32),
            pltpu.VMEM((CHUNK, D), jnp.float32),
            pltpu.VMEM_SHARED((n_pad, D), jnp.float32),
            pltpu.SemaphoreType.DMA,
            pltpu.SemaphoreType.DMA,
            pltpu.SemaphoreType.DMA,
        ],
    )
    def agg_kernel(m_hbm, src2d, dst2d, out, sidx, didx, buf0, buf1,
                   shared, sem, gsem, ssem):
        bufs = (buf0, buf1)
        c = lax.axis_index("c")
        s = lax.axis_index("s")
        wbase = s * nw

        def load_idx(w):
            base = (wbase + w) * H
            pltpu.async_copy(src2d.at[pl.ds(base, H)], sidx, sem)
            pltpu.async_copy(dst2d.at[pl.ds(base, H)], didx, sem)
            pltpu.make_async_copy(src2d.at[pl.ds(base, H)], sidx, sem).wait()
            pltpu.make_async_copy(dst2d.at[pl.ds(base, H)], didx, sem).wait()

        def start_g(j, r):
            pltpu.async_copy(m_hbm.at[sidx.at[j]], bufs[r], gsem)

        def start_s(j, r):
            pltpu.async_copy(bufs[r], shared.at[didx.at[j]], ssem, add=True)

        def wait_g(j, r):
            pltpu.make_async_copy(m_hbm.at[sidx.at[j]], bufs[r], gsem).wait()

        def wait_s(j, r):
            pltpu.make_async_copy(bufs[r], shared.at[didx.at[j]],
                                  ssem).wait()

        @pl.when(c == 0)
        def _run():
            cp_i = pltpu.async_copy(m_hbm.at[pl.ds(s * rpt, rpt)],
                                    shared.at[pl.ds(s * rpt, rpt)], gsem)
            load_idx(0)
            cp_i.wait()
            plsc.subcore_barrier()

            def body(g, _):
                for r in (0, 1):
                    j = 2 * g + r
                    wait_g(j, r)
                    start_s(j, r)
                    wait_s(j - 1, 1 - r)

                    @pl.when(j + 1 < H)
                    def _():
                        start_g(j + 1, 1 - r)

                return 0

            def run_window(w, _):
                @pl.when(w > 0)
                def _():
                    load_idx(w)

                start_g(0, 0)
                wait_g(0, 0)
                start_s(0, 0)
                start_g(1, 1)
                wait_g(1, 1)
                start_s(1, 1)
                wait_s(0, 0)
                start_g(2, 0)
                lax.fori_loop(1, H // 2, body, 0)
                wait_s(H - 1, 1)
                return 0

            lax.fori_loop(0, nw, run_window, 0)
            plsc.subcore_barrier()
            pltpu.sync_copy(shared.at[pl.ds(s * rpt, rpt)],
                            out.at[0, pl.ds(s * rpt, rpt)])

    return agg_kernel


def _bdot(a, b):
    return jnp.dot(a.astype(jnp.bfloat16), b.astype(jnp.bfloat16),
                   preferred_element_type=jnp.float32)


def _dinv_from(degp_ref):
    deg = degp_ref[0, :] + degp_ref[1, :] + 1.0
    return lax.rsqrt(deg)


def _tc_first(degp, xp, W0, b0, W1):
    n_pad = xp.shape[0]

    def body(degp_ref, x_ref, w0_ref, b0_ref, w1_ref, m_ref):
        dinv = _dinv_from(degp_ref)
        h0 = _bdot(x_ref[...], w0_ref[...]) + b0_ref[...][None, :]
        m_ref[...] = dinv[:, None] * _bdot(h0, w1_ref[...])

    return pl.pallas_call(
        body,
        grid=(n_pad // BLK,),
        in_specs=[
            pl.BlockSpec((2, BLK), lambda i: (0, i)),
            pl.BlockSpec((BLK, D), lambda i: (i, 0)),
            pl.BlockSpec((D, D), lambda i: (0, 0)),
            pl.BlockSpec((D,), lambda i: (0,)),
            pl.BlockSpec((D, D), lambda i: (0, 0)),
        ],
        out_specs=pl.BlockSpec((BLK, D), lambda i: (i, 0)),
        out_shape=jax.ShapeDtypeStruct((n_pad, D), jnp.float32),
    )(degp, xp, W0, b0, W1)


def _tc_mid(degp, part, b_prev, W_next):
    n_pad = part.shape[1]

    def body(degp_ref, p_ref, b_ref, w_ref, o_ref):
        dinv = _dinv_from(degp_ref)
        h = jax.nn.relu(dinv[:, None] * p_ref[0] + b_ref[...][None, :])
        o_ref[...] = dinv[:, None] * _bdot(h, w_ref[...])

    return pl.pallas_call(
        body,
        grid=(n_pad // BLK,),
        in_specs=[
            pl.BlockSpec((2, BLK), lambda i: (0, i)),
            pl.BlockSpec((1, BLK, D), lambda i: (0, i, 0)),
            pl.BlockSpec((D,), lambda i: (0,)),
            pl.BlockSpec((D, D), lambda i: (0, 0)),
        ],
        out_specs=pl.BlockSpec((BLK, D), lambda i: (i, 0)),
        out_shape=jax.ShapeDtypeStruct((n_pad, D), jnp.float32),
    )(degp, part, b_prev, W_next)


def _tc_final(degp, part, b3, wf_row, bf2):
    n_pad = part.shape[1]

    def body(degp_ref, p_ref, b_ref, wf_ref, bf_ref, o_ref):
        dinv = _dinv_from(degp_ref)
        h3 = dinv[:, None] * p_ref[0] + b_ref[...][None, :]
        h3r = h3.astype(jnp.bfloat16).astype(jnp.float32)
        wfr = wf_ref[...].astype(jnp.bfloat16).astype(jnp.float32)
        o_ref[...] = jnp.sum(h3r * wfr, axis=1) + bf_ref[0]

    return pl.pallas_call(
        body,
        grid=(n_pad // BLK,),
        in_specs=[
            pl.BlockSpec((2, BLK), lambda i: (0, i)),
            pl.BlockSpec((1, BLK, D), lambda i: (0, i, 0)),
            pl.BlockSpec((D,), lambda i: (0,)),
            pl.BlockSpec((1, D), lambda i: (0, 0)),
            pl.BlockSpec(memory_space=pltpu.MemorySpace.SMEM),
        ],
        out_specs=pl.BlockSpec((BLK,), lambda i: (i,)),
        out_shape=jax.ShapeDtypeStruct((n_pad,), jnp.float32),
    )(degp, part, b3, wf_row, bf2)


def kernel(x, edge_index, W0, b0, W1, b1, W2, b2, W3, b3, Wf, bf):
    n = x.shape[0]
    e = edge_index.shape[1]
    n_pad = ((n + BLK - 1) // BLK) * BLK
    gran = NW * CHUNK * 8
    e_pad = ((e + gran - 1) // gran) * gran
    jpt = e_pad // (NW * CHUNK)

    npad_ids = jnp.arange(e_pad - e, dtype=jnp.int32)
    pad_src = jnp.full((e_pad - e,), n, jnp.int32)
    pad_dst = n + (npad_ids % (n_pad - n))
    src2d = jnp.concatenate([edge_index[0], pad_src]).reshape(
        e_pad // CHUNK, CHUNK)
    dst2d = jnp.concatenate([edge_index[1], pad_dst]).reshape(
        e_pad // CHUNK, CHUNK)
    xp = jnp.pad(x, ((0, n_pad - n), (0, 0)))
    wf_row = Wf.reshape(1, D)
    bf2 = bf.reshape(1)

    deg_k = _make_deg_kernel(n_pad, jpt)
    agg_k = _make_agg_kernel(n_pad, jpt, 4)

    degp = deg_k(dst2d)
    m1 = _tc_first(degp, xp, W0, b0, W1)
    p1 = agg_k(m1, src2d, dst2d)
    m2 = _tc_mid(degp, p1, b1, W2)
    p2 = agg_k(m2, src2d, dst2d)
    m3 = _tc_mid(degp, p2, b2, W3)
    p3 = agg_k(m3, src2d, dst2d)
    outp = _tc_final(degp, p3, b3, wf_row, bf2)
    return outp[:n]


# speedup vs baseline: 1.0624x; 1.0540x over previous
import functools

import jax
import jax.numpy as jnp
from jax import lax
from jax.experimental import pallas as pl
from jax.experimental.pallas import tpu as pltpu
from jax.experimental.pallas import tpu_sc as plsc

NC = 2
NS = 16
NW = NC * NS
D = 128
BLK = 512
CHUNK = 128


def _sc_mesh():
    return plsc.VectorSubcoreMesh(core_axis_name="c", subcore_axis_name="s")


def _make_deg_kernel(n_pad, jpt):
    rpt = n_pad // NS

    @functools.partial(
        pl.kernel,
        out_type=jax.ShapeDtypeStruct((NC, n_pad), jnp.float32),
        mesh=_sc_mesh(),
        scratch_types=[
            pltpu.VMEM((jpt, CHUNK), jnp.int32),
            pltpu.VMEM((CHUNK,), jnp.float32),
            pltpu.VMEM((rpt,), jnp.float32),
            pltpu.VMEM_SHARED((n_pad,), jnp.float32),
        ],
    )
    def deg_kernel(dst2d, out, idx_v, ones_v, zeros_v, shared):
        c = lax.axis_index("c")
        s = lax.axis_index("s")
        wid = c * NS + s
        pltpu.sync_copy(dst2d.at[pl.ds(wid * jpt, jpt)], idx_v)

        def fill_ones(i, _):
            ones_v[pl.ds(i * 16, 16)] = jnp.full((16,), 1.0, jnp.float32)
            return 0

        lax.fori_loop(0, CHUNK // 16, fill_ones, 0)

        def fill_zeros(i, _):
            zeros_v[pl.ds(i * 16, 16)] = jnp.zeros((16,), jnp.float32)
            return 0

        lax.fori_loop(0, rpt // 16, fill_zeros, 0)
        pltpu.sync_copy(zeros_v, shared.at[pl.ds(s * rpt, rpt)])
        plsc.subcore_barrier()

        def body(j, _):
            pltpu.sync_copy(ones_v, shared.at[idx_v.at[j]], add=True)
            return 0

        lax.fori_loop(0, jpt, body, 0)
        plsc.subcore_barrier()
        pltpu.sync_copy(shared.at[pl.ds(s * rpt, rpt)],
                        out.at[c, pl.ds(s * rpt, rpt)])

    return deg_kernel


def _make_agg_kernel(n_pad, jpt, nw0, nw1):
    rpt = n_pad // NS

    H = jpt * 2 // (nw0 + nw1)
    assert H % 8 == 0 and H >= 4

    @functools.partial(
        pl.kernel,
        out_type=jax.ShapeDtypeStruct((NC, n_pad, D), jnp.float32),
        mesh=_sc_mesh(),
        scratch_types=[
            pltpu.VMEM((H, CHUNK), jnp.int32),
            pltpu.VMEM((H, CHUNK), jnp.int32),
            pltpu.VMEM((CHUNK, D), jnp.float32),
            pltpu.VMEM((CHUNK, D), jnp.float32),
            pltpu.VMEM_SHARED((n_pad, D), jnp.float32),
            pltpu.SemaphoreType.DMA,
            pltpu.SemaphoreType.DMA,
            pltpu.SemaphoreType.DMA,
        ],
    )
    def agg_kernel(m_hbm, src2d, dst2d, out, sidx, didx, buf0, buf1,
                   shared, sem, gsem, ssem):
        bufs = (buf0, buf1)
        c = lax.axis_index("c")
        s = lax.axis_index("s")
        nw = jnp.where(c == 0, nw0, nw1)
        wbase = jnp.where(c == 0, s * nw0, NS * nw0 + s * nw1)

        def load_idx(w):
            base = (wbase + w) * H
            pltpu.async_copy(src2d.at[pl.ds(base, H)], sidx, sem)
            pltpu.async_copy(dst2d.at[pl.ds(base, H)], didx, sem)
            pltpu.make_async_copy(src2d.at[pl.ds(base, H)], sidx, sem).wait()
            pltpu.make_async_copy(dst2d.at[pl.ds(base, H)], didx, sem).wait()

        def start_g(j, r):
            pltpu.async_copy(m_hbm.at[sidx.at[j]], bufs[r], gsem)

        def start_s(j, r):
            pltpu.async_copy(bufs[r], shared.at[didx.at[j]], ssem, add=True)

        def wait_g(j, r):
            pltpu.make_async_copy(m_hbm.at[sidx.at[j]], bufs[r], gsem).wait()

        def wait_s(j, r):
            pltpu.make_async_copy(bufs[r], shared.at[didx.at[j]],
                                  ssem).wait()

        @pl.when(c == 0)
        def _init_m():
            cp_i = pltpu.async_copy(m_hbm.at[pl.ds(s * rpt, rpt)],
                                    shared.at[pl.ds(s * rpt, rpt)], gsem)
            load_idx(0)
            cp_i.wait()

        @pl.when(c == 1)
        def _init_zero():
            def zrow(i, _):
                for k in range(D // 16):
                    buf0[i, pl.ds(k * 16, 16)] = jnp.zeros((16,),
                                                           jnp.float32)
                return 0

            lax.fori_loop(0, CHUNK, zrow, 0)
            for t in range(rpt // CHUNK):
                pltpu.sync_copy(
                    buf0, shared.at[pl.ds(s * rpt + t * CHUNK, CHUNK)])
            load_idx(0)

        plsc.subcore_barrier()
        if True:

            def body(g, _):
                for r in (0, 1):
                    j = 2 * g + r
                    wait_g(j, r)
                    start_s(j, r)
                    wait_s(j - 1, 1 - r)

                    @pl.when(j + 1 < H)
                    def _():
                        start_g(j + 1, 1 - r)

                return 0

            def run_window(w, _):
                @pl.when(w > 0)
                def _():
                    load_idx(w)

                start_g(0, 0)
                wait_g(0, 0)
                start_s(0, 0)
                start_g(1, 1)
                wait_g(1, 1)
                start_s(1, 1)
                wait_s(0, 0)
                start_g(2, 0)
                lax.fori_loop(1, H // 2, body, 0)
                wait_s(H - 1, 1)
                return 0

            lax.fori_loop(0, nw, run_window, 0)
            plsc.subcore_barrier()
            pltpu.sync_copy(shared.at[pl.ds(s * rpt, rpt)],
                            out.at[c, pl.ds(s * rpt, rpt)])

    return agg_kernel


def _bdot(a, b):
    return jnp.dot(a.astype(jnp.bfloat16), b.astype(jnp.bfloat16),
                   preferred_element_type=jnp.float32)


def _dinv_from(degp_ref):
    deg = degp_ref[0, :] + degp_ref[1, :] + 1.0
    return lax.rsqrt(deg)


def _tc_first(degp, xp, W0, b0, W1):
    n_pad = xp.shape[0]

    def body(degp_ref, x_ref, w0_ref, b0_ref, w1_ref, m_ref):
        dinv = _dinv_from(degp_ref)
        h0 = _bdot(x_ref[...], w0_ref[...]) + b0_ref[...][None, :]
        m_ref[...] = dinv[:, None] * _bdot(h0, w1_ref[...])

    return pl.pallas_call(
        body,
        grid=(n_pad // BLK,),
        in_specs=[
            pl.BlockSpec((2, BLK), lambda i: (0, i)),
            pl.BlockSpec((BLK, D), lambda i: (i, 0)),
            pl.BlockSpec((D, D), lambda i: (0, 0)),
            pl.BlockSpec((D,), lambda i: (0,)),
            pl.BlockSpec((D, D), lambda i: (0, 0)),
        ],
        out_specs=pl.BlockSpec((BLK, D), lambda i: (i, 0)),
        out_shape=jax.ShapeDtypeStruct((n_pad, D), jnp.float32),
    )(degp, xp, W0, b0, W1)


def _tc_mid(degp, part, b_prev, W_next):
    n_pad = part.shape[1]

    def body(degp_ref, p_ref, b_ref, w_ref, o_ref):
        dinv = _dinv_from(degp_ref)
        agg = p_ref[0] + p_ref[1]
        h = jax.nn.relu(dinv[:, None] * agg + b_ref[...][None, :])
        o_ref[...] = dinv[:, None] * _bdot(h, w_ref[...])

    return pl.pallas_call(
        body,
        grid=(n_pad // BLK,),
        in_specs=[
            pl.BlockSpec((2, BLK), lambda i: (0, i)),
            pl.BlockSpec((2, BLK, D), lambda i: (0, i, 0)),
            pl.BlockSpec((D,), lambda i: (0,)),
            pl.BlockSpec((D, D), lambda i: (0, 0)),
        ],
        out_specs=pl.BlockSpec((BLK, D), lambda i: (i, 0)),
        out_shape=jax.ShapeDtypeStruct((n_pad, D), jnp.float32),
    )(degp, part, b_prev, W_next)


def _tc_final(degp, part, b3, wf_row, bf2):
    n_pad = part.shape[1]

    def body(degp_ref, p_ref, b_ref, wf_ref, bf_ref, o_ref):
        dinv = _dinv_from(degp_ref)
        h3 = dinv[:, None] * (p_ref[0] + p_ref[1]) + b_ref[...][None, :]
        h3r = h3.astype(jnp.bfloat16).astype(jnp.float32)
        wfr = wf_ref[...].astype(jnp.bfloat16).astype(jnp.float32)
        o_ref[...] = jnp.sum(h3r * wfr, axis=1) + bf_ref[0]

    return pl.pallas_call(
        body,
        grid=(n_pad // BLK,),
        in_specs=[
            pl.BlockSpec((2, BLK), lambda i: (0, i)),
            pl.BlockSpec((2, BLK, D), lambda i: (0, i, 0)),
            pl.BlockSpec((D,), lambda i: (0,)),
            pl.BlockSpec((1, D), lambda i: (0, 0)),
            pl.BlockSpec(memory_space=pltpu.MemorySpace.SMEM),
        ],
        out_specs=pl.BlockSpec((BLK,), lambda i: (i,)),
        out_shape=jax.ShapeDtypeStruct((n_pad,), jnp.float32),
    )(degp, part, b3, wf_row, bf2)


def kernel(x, edge_index, W0, b0, W1, b1, W2, b2, W3, b3, Wf, bf):
    n = x.shape[0]
    e = edge_index.shape[1]
    n_pad = ((n + BLK - 1) // BLK) * BLK
    gran = NW * CHUNK * 8
    e_pad = ((e + gran - 1) // gran) * gran
    jpt = e_pad // (NW * CHUNK)

    npad_ids = jnp.arange(e_pad - e, dtype=jnp.int32)
    pad_src = jnp.full((e_pad - e,), n, jnp.int32)
    pad_dst = n + (npad_ids % (n_pad - n))
    src2d = jnp.concatenate([edge_index[0], pad_src]).reshape(
        e_pad // CHUNK, CHUNK)
    dst2d = jnp.concatenate([edge_index[1], pad_dst]).reshape(
        e_pad // CHUNK, CHUNK)
    xp = jnp.pad(x, ((0, n_pad - n), (0, 0)))
    wf_row = Wf.reshape(1, D)
    bf2 = bf.reshape(1)

    deg_k = _make_deg_kernel(n_pad, jpt)
    agg_k = _make_agg_kernel(n_pad, jpt, 3, 1)

    degp = deg_k(dst2d)
    m1 = _tc_first(degp, xp, W0, b0, W1)
    p1 = agg_k(m1, src2d, dst2d)
    m2 = _tc_mid(degp, p1, b1, W2)
    p2 = agg_k(m2, src2d, dst2d)
    m3 = _tc_mid(degp, p2, b2, W3)
    p3 = agg_k(m3, src2d, dst2d)
    outp = _tc_final(degp, p3, b3, wf_row, bf2)
    return outp[:n]


# speedup vs baseline: 1.1014x; 1.0367x over previous
import functools

import jax
import jax.numpy as jnp
from jax import lax
from jax.experimental import pallas as pl
from jax.experimental.pallas import tpu as pltpu
from jax.experimental.pallas import tpu_sc as plsc

NC = 2
NS = 16
NW = NC * NS
D = 128
BLK = 512
CHUNK = 128


def _sc_mesh():
    return plsc.VectorSubcoreMesh(core_axis_name="c", subcore_axis_name="s")


def _make_deg_kernel(n_pad, jpt):
    rpt = n_pad // NS

    @functools.partial(
        pl.kernel,
        out_type=jax.ShapeDtypeStruct((NC, n_pad), jnp.float32),
        mesh=_sc_mesh(),
        scratch_types=[
            pltpu.VMEM((jpt, CHUNK), jnp.int32),
            pltpu.VMEM((CHUNK,), jnp.float32),
            pltpu.VMEM((rpt,), jnp.float32),
            pltpu.VMEM_SHARED((n_pad,), jnp.float32),
        ],
    )
    def deg_kernel(dst2d, out, idx_v, ones_v, zeros_v, shared):
        c = lax.axis_index("c")
        s = lax.axis_index("s")
        wid = c * NS + s
        pltpu.sync_copy(dst2d.at[pl.ds(wid * jpt, jpt)], idx_v)

        def fill_ones(i, _):
            ones_v[pl.ds(i * 16, 16)] = jnp.full((16,), 1.0, jnp.float32)
            return 0

        lax.fori_loop(0, CHUNK // 16, fill_ones, 0)

        def fill_zeros(i, _):
            zeros_v[pl.ds(i * 16, 16)] = jnp.zeros((16,), jnp.float32)
            return 0

        lax.fori_loop(0, rpt // 16, fill_zeros, 0)
        pltpu.sync_copy(zeros_v, shared.at[pl.ds(s * rpt, rpt)])
        plsc.subcore_barrier()

        def body(j, _):
            pltpu.sync_copy(ones_v, shared.at[idx_v.at[j]], add=True)
            return 0

        lax.fori_loop(0, jpt, body, 0)
        plsc.subcore_barrier()
        pltpu.sync_copy(shared.at[pl.ds(s * rpt, rpt)],
                        out.at[c, pl.ds(s * rpt, rpt)])

    return deg_kernel


def _make_agg_kernel(n_pad, jpt, nw0, nw1):
    rpt = n_pad // NS

    H = jpt * 2 // (nw0 + nw1)
    assert H % 8 == 0 and H >= 4

    @functools.partial(
        pl.kernel,
        out_type=jax.ShapeDtypeStruct((NC, n_pad, D), jnp.float32),
        mesh=_sc_mesh(),
        scratch_types=[
            pltpu.VMEM((H, CHUNK), jnp.int32),
            pltpu.VMEM((H, CHUNK), jnp.int32),
            pltpu.VMEM((CHUNK, D), jnp.float32),
            pltpu.VMEM((CHUNK, D), jnp.float32),
            pltpu.VMEM_SHARED((n_pad, D), jnp.float32),
            pltpu.SemaphoreType.DMA,
            pltpu.SemaphoreType.DMA,
            pltpu.SemaphoreType.DMA,
        ],
    )
    def agg_kernel(m_hbm, src2d, dst2d, out, sidx, didx, buf0, buf1,
                   shared, sem, gsem, ssem):
        bufs = (buf0, buf1)
        c = lax.axis_index("c")
        s = lax.axis_index("s")
        nw = jnp.where(c == 0, nw0, nw1)
        wbase = jnp.where(c == 0, s * nw0, NS * nw0 + s * nw1)

        def load_idx(w):
            base = (wbase + w) * H
            pltpu.async_copy(src2d.at[pl.ds(base, H)], sidx, sem)
            pltpu.async_copy(dst2d.at[pl.ds(base, H)], didx, sem)
            pltpu.make_async_copy(src2d.at[pl.ds(base, H)], sidx, sem).wait()
            pltpu.make_async_copy(dst2d.at[pl.ds(base, H)], didx, sem).wait()

        def start_g(j, r):
            pltpu.async_copy(m_hbm.at[sidx.at[j]], bufs[r], gsem)

        def start_s(j, r):
            pltpu.async_copy(bufs[r], shared.at[didx.at[j]], ssem, add=True)

        def wait_g(j, r):
            pltpu.make_async_copy(m_hbm.at[sidx.at[j]], bufs[r], gsem).wait()

        def wait_s(j, r):
            pltpu.make_async_copy(bufs[r], shared.at[didx.at[j]],
                                  ssem).wait()

        cp_i = pltpu.async_copy(m_hbm.at[pl.ds(s * rpt, rpt)],
                                shared.at[pl.ds(s * rpt, rpt)], gsem)
        load_idx(0)
        cp_i.wait()
        plsc.subcore_barrier()
        if True:

            def body(g, _):
                for r in (0, 1):
                    j = 2 * g + r
                    wait_g(j, r)
                    start_s(j, r)
                    wait_s(j - 1, 1 - r)

                    @pl.when(j + 1 < H)
                    def _():
                        start_g(j + 1, 1 - r)

                return 0

            def run_window(w, _):
                @pl.when(w > 0)
                def _():
                    load_idx(w)

                start_g(0, 0)
                wait_g(0, 0)
                start_s(0, 0)
                start_g(1, 1)
                wait_g(1, 1)
                start_s(1, 1)
                wait_s(0, 0)
                start_g(2, 0)
                lax.fori_loop(1, H // 2, body, 0)
                wait_s(H - 1, 1)
                return 0

            lax.fori_loop(0, nw, run_window, 0)
            plsc.subcore_barrier()
            pltpu.sync_copy(shared.at[pl.ds(s * rpt, rpt)],
                            out.at[c, pl.ds(s * rpt, rpt)])

    return agg_kernel


def _bdot(a, b):
    return jnp.dot(a.astype(jnp.bfloat16), b.astype(jnp.bfloat16),
                   preferred_element_type=jnp.float32)


def _dinv_from(degp_ref):
    deg = degp_ref[0, :] + degp_ref[1, :] + 1.0
    return lax.rsqrt(deg)


def _tc_first(degp, xp, W0, b0, W1):
    n_pad = xp.shape[0]

    def body(degp_ref, x_ref, w0_ref, b0_ref, w1_ref, m_ref):
        dinv = _dinv_from(degp_ref)
        h0 = _bdot(x_ref[...], w0_ref[...]) + b0_ref[...][None, :]
        m_ref[...] = dinv[:, None] * _bdot(h0, w1_ref[...])

    return pl.pallas_call(
        body,
        grid=(n_pad // BLK,),
        in_specs=[
            pl.BlockSpec((2, BLK), lambda i: (0, i)),
            pl.BlockSpec((BLK, D), lambda i: (i, 0)),
            pl.BlockSpec((D, D), lambda i: (0, 0)),
            pl.BlockSpec((D,), lambda i: (0,)),
            pl.BlockSpec((D, D), lambda i: (0, 0)),
        ],
        out_specs=pl.BlockSpec((BLK, D), lambda i: (i, 0)),
        out_shape=jax.ShapeDtypeStruct((n_pad, D), jnp.float32),
    )(degp, xp, W0, b0, W1)


def _tc_mid(degp, part, m_prev, b_prev, W_next):
    n_pad = part.shape[1]

    def body(degp_ref, p_ref, m_ref, b_ref, w_ref, o_ref):
        dinv = _dinv_from(degp_ref)
        agg = p_ref[0] + p_ref[1] - m_ref[...]
        h = jax.nn.relu(dinv[:, None] * agg + b_ref[...][None, :])
        o_ref[...] = dinv[:, None] * _bdot(h, w_ref[...])

    return pl.pallas_call(
        body,
        grid=(n_pad // BLK,),
        in_specs=[
            pl.BlockSpec((2, BLK), lambda i: (0, i)),
            pl.BlockSpec((2, BLK, D), lambda i: (0, i, 0)),
            pl.BlockSpec((BLK, D), lambda i: (i, 0)),
            pl.BlockSpec((D,), lambda i: (0,)),
            pl.BlockSpec((D, D), lambda i: (0, 0)),
        ],
        out_specs=pl.BlockSpec((BLK, D), lambda i: (i, 0)),
        out_shape=jax.ShapeDtypeStruct((n_pad, D), jnp.float32),
    )(degp, part, m_prev, b_prev, W_next)


def _tc_final(degp, part, m3, b3, wf_row, bf2):
    n_pad = part.shape[1]

    def body(degp_ref, p_ref, m_ref, b_ref, wf_ref, bf_ref, o_ref):
        dinv = _dinv_from(degp_ref)
        h3 = (dinv[:, None] * (p_ref[0] + p_ref[1] - m_ref[...])
              + b_ref[...][None, :])
        h3r = h3.astype(jnp.bfloat16).astype(jnp.float32)
        wfr = wf_ref[...].astype(jnp.bfloat16).astype(jnp.float32)
        o_ref[...] = jnp.sum(h3r * wfr, axis=1) + bf_ref[0]

    return pl.pallas_call(
        body,
        grid=(n_pad // BLK,),
        in_specs=[
            pl.BlockSpec((2, BLK), lambda i: (0, i)),
            pl.BlockSpec((2, BLK, D), lambda i: (0, i, 0)),
            pl.BlockSpec((BLK, D), lambda i: (i, 0)),
            pl.BlockSpec((D,), lambda i: (0,)),
            pl.BlockSpec((1, D), lambda i: (0, 0)),
            pl.BlockSpec(memory_space=pltpu.MemorySpace.SMEM),
        ],
        out_specs=pl.BlockSpec((BLK,), lambda i: (i,)),
        out_shape=jax.ShapeDtypeStruct((n_pad,), jnp.float32),
    )(degp, part, m3, b3, wf_row, bf2)


def kernel(x, edge_index, W0, b0, W1, b1, W2, b2, W3, b3, Wf, bf):
    n = x.shape[0]
    e = edge_index.shape[1]
    n_pad = ((n + BLK - 1) // BLK) * BLK
    gran = NW * CHUNK * 8
    e_pad = ((e + gran - 1) // gran) * gran
    jpt = e_pad // (NW * CHUNK)

    npad_ids = jnp.arange(e_pad - e, dtype=jnp.int32)
    pad_src = jnp.full((e_pad - e,), n, jnp.int32)
    pad_dst = n + (npad_ids % (n_pad - n))
    src2d = jnp.concatenate([edge_index[0], pad_src]).reshape(
        e_pad // CHUNK, CHUNK)
    dst2d = jnp.concatenate([edge_index[1], pad_dst]).reshape(
        e_pad // CHUNK, CHUNK)
    xp = jnp.pad(x, ((0, n_pad - n), (0, 0)))
    wf_row = Wf.reshape(1, D)
    bf2 = bf.reshape(1)

    deg_k = _make_deg_kernel(n_pad, jpt)
    agg_k = _make_agg_kernel(n_pad, jpt, 3, 1)

    degp = deg_k(dst2d)
    m1 = _tc_first(degp, xp, W0, b0, W1)
    p1 = agg_k(m1, src2d, dst2d)
    m2 = _tc_mid(degp, p1, m1, b1, W2)
    p2 = agg_k(m2, src2d, dst2d)
    m3 = _tc_mid(degp, p2, m2, b2, W3)
    p3 = agg_k(m3, src2d, dst2d)
    outp = _tc_final(degp, p3, m3, b3, wf_row, bf2)
    return outp[:n]
